# baseline (speedup 1.0000x reference)
import jax
import jax.numpy as jnp
from jax.experimental import pallas as pl

_Y_LOC = 100
_X_LOC = 100
_T_H = 96
_T_W = 96


def _erase_body(x_ref, o_ref):
    o_ref[...] = x_ref[...].astype(jnp.uint8)
    o_ref[0, _Y_LOC:_Y_LOC + _T_H, _X_LOC * 3:(_X_LOC + _T_W) * 3] = (
        jnp.zeros((_T_H, _T_W * 3), jnp.uint8))


def kernel(inputs):
    b, h, w, c = inputs.shape
    x = inputs.reshape(b, h, w * c)
    out = pl.pallas_call(
        _erase_body,
        grid=(b,),
        in_specs=[pl.BlockSpec((1, h, w * c), lambda i: (i, 0, 0))],
        out_specs=pl.BlockSpec((1, h, w * c), lambda i: (i, 0, 0)),
        out_shape=jax.ShapeDtypeStruct((b, h, w * c), jnp.uint8),
    )(x)
    return out.reshape(b, h, w, c)


# speedup vs baseline: 4.1761x; 4.1761x over previous
import jax
import jax.numpy as jnp
from jax.experimental import pallas as pl

_Y_LOC = 100
_X_LOC = 100
_T_H = 96
_T_W = 96


def _erase_body(x_ref, o_ref):
    o_ref[...] = x_ref[...].astype(jnp.uint8)
    o_ref[0, _Y_LOC:_Y_LOC + _T_H, _X_LOC:_X_LOC + _T_W] = (
        jnp.zeros((_T_H, _T_W), jnp.uint8))


def kernel(inputs):
    b, h, w, c = inputs.shape
    x = jnp.transpose(inputs, (0, 3, 1, 2)).reshape(b * c, h, w)
    out = pl.pallas_call(
        _erase_body,
        grid=(b * c,),
        in_specs=[pl.BlockSpec((1, h, w), lambda i: (i, 0, 0))],
        out_specs=pl.BlockSpec((1, h, w), lambda i: (i, 0, 0)),
        out_shape=jax.ShapeDtypeStruct((b * c, h, w), jnp.uint8),
    )(x)
    return jnp.transpose(out.reshape(b, c, h, w), (0, 2, 3, 1))


# speedup vs baseline: 9.3062x; 2.2284x over previous
import jax
import jax.numpy as jnp
from jax.experimental import pallas as pl

_Y_LOC = 100
_X_LOC = 100
_T_H = 96
_T_W = 96


_P = 4


def _erase_body(x_ref, o_ref):
    o_ref[...] = x_ref[...].astype(jnp.uint8)
    o_ref[:, _Y_LOC:_Y_LOC + _T_H, _X_LOC:_X_LOC + _T_W] = (
        jnp.zeros((_P, _T_H, _T_W), jnp.uint8))


def kernel(inputs):
    b, h, w, c = inputs.shape
    x = jnp.transpose(inputs, (0, 3, 1, 2)).reshape(b * c, h, w)
    out = pl.pallas_call(
        _erase_body,
        grid=(b * c // _P,),
        in_specs=[pl.BlockSpec((_P, h, w), lambda i: (i, 0, 0))],
        out_specs=pl.BlockSpec((_P, h, w), lambda i: (i, 0, 0)),
        out_shape=jax.ShapeDtypeStruct((b * c, h, w), jnp.uint8),
    )(x)
    return jnp.transpose(out.reshape(b, c, h, w), (0, 2, 3, 1))


# speedup vs baseline: 11.1870x; 1.2021x over previous
import jax
import jax.numpy as jnp
from jax.experimental import pallas as pl

_Y_LOC = 100
_X_LOC = 100
_T_H = 96
_T_W = 96


_P = 8


def _erase_body(x_ref, o_ref):
    o_ref[...] = x_ref[...].astype(jnp.uint8)
    o_ref[:, _Y_LOC:_Y_LOC + _T_H, _X_LOC:_X_LOC + _T_W] = (
        jnp.zeros((_P, _T_H, _T_W), jnp.uint8))


def kernel(inputs):
    b, h, w, c = inputs.shape
    x = jnp.transpose(inputs, (0, 3, 1, 2)).reshape(b * c, h, w)
    out = pl.pallas_call(
        _erase_body,
        grid=(b * c // _P,),
        in_specs=[pl.BlockSpec((_P, h, w), lambda i: (i, 0, 0))],
        out_specs=pl.BlockSpec((_P, h, w), lambda i: (i, 0, 0)),
        out_shape=jax.ShapeDtypeStruct((b * c, h, w), jnp.uint8),
    )(x)
    return jnp.transpose(out.reshape(b, c, h, w), (0, 2, 3, 1))


# speedup vs baseline: 11.2798x; 1.0083x over previous
import jax
import jax.numpy as jnp
from jax.experimental import pallas as pl

_Y_LOC = 100
_X_LOC = 100
_T_H = 96
_T_W = 96


_P = 16


def _erase_body(x_ref, o_ref):
    o_ref[...] = x_ref[...].astype(jnp.uint8)
    o_ref[:, _Y_LOC:_Y_LOC + _T_H, _X_LOC:_X_LOC + _T_W] = (
        jnp.zeros((_P, _T_H, _T_W), jnp.uint8))


def kernel(inputs):
    b, h, w, c = inputs.shape
    x = jnp.transpose(inputs, (0, 3, 1, 2)).reshape(b * c, h, w)
    out = pl.pallas_call(
        _erase_body,
        grid=(b * c // _P,),
        in_specs=[pl.BlockSpec((_P, h, w), lambda i: (i, 0, 0))],
        out_specs=pl.BlockSpec((_P, h, w), lambda i: (i, 0, 0)),
        out_shape=jax.ShapeDtypeStruct((b * c, h, w), jnp.uint8),
    )(x)
    return jnp.transpose(out.reshape(b, c, h, w), (0, 2, 3, 1))
